# Initial kernel scaffold; baseline (speedup 1.0000x reference)
import jax
import jax.numpy as jnp
from jax.experimental import pallas as pl


def kernel(positions):
    raise NotImplementedError("write your pallas kernel here")


# speedup vs baseline: 9.2618x; 9.2618x over previous
import functools

import jax
import jax.numpy as jnp
from jax.experimental import pallas as pl

K = 16
BM = 256


def _fused_kernel(p_blk_ref, p_all_ref, idx_ref, w_ref):
    pb = p_blk_ref[0]
    pa = p_all_ref[0]
    n = pa.shape[0]

    inner = jnp.dot(pb, pa.T, preferred_element_type=jnp.float32)
    x2b = jnp.sum(pb * pb, axis=1)
    x2a = jnp.sum(pa * pa, axis=1)
    d2 = x2b[:, None] + x2a[None, :] - 2.0 * inner
    d2 = jnp.maximum(d2, 1e-12)

    i = pl.program_id(1)
    row_ids = i * BM + jax.lax.broadcasted_iota(jnp.int32, (BM, n), 0)
    col_ids = jax.lax.broadcasted_iota(jnp.int32, (BM, n), 1)
    d2 = jnp.where(row_ids == col_ids, jnp.inf, d2)

    vals = []
    idxs = []
    work = d2
    for _ in range(K):
        m = jnp.min(work, axis=1)
        hit = work == m[:, None]
        j = jnp.min(jnp.where(hit, col_ids, n), axis=1)
        work = jnp.where(col_ids == j[:, None], jnp.inf, work)
        vals.append(m)
        idxs.append(j)

    v = jnp.stack(vals, axis=1)
    dist = jnp.sqrt(v)
    scores = -dist
    e = jnp.exp(scores - scores[:, :1])
    w = e / jnp.sum(e, axis=1, keepdims=True)

    idx_ref[0] = jnp.stack(idxs, axis=1)
    w_ref[0] = w


@jax.jit
def kernel(positions):
    batch, agents, dim = positions.shape
    grid = (batch, agents // BM)

    out_shape = (
        jax.ShapeDtypeStruct((batch, agents, K), jnp.int32),
        jax.ShapeDtypeStruct((batch, agents, K), jnp.float32),
    )
    idx, w = pl.pallas_call(
        _fused_kernel,
        grid=grid,
        in_specs=[
            pl.BlockSpec((1, BM, dim), lambda b, i: (b, i, 0)),
            pl.BlockSpec((1, agents, dim), lambda b, i: (b, 0, 0)),
        ],
        out_specs=[
            pl.BlockSpec((1, BM, K), lambda b, i: (b, i, 0)),
            pl.BlockSpec((1, BM, K), lambda b, i: (b, i, 0)),
        ],
        out_shape=out_shape,
    )(positions, positions)
    return idx, w


# speedup vs baseline: 12.3173x; 1.3299x over previous
import functools

import jax
import jax.numpy as jnp
from jax.experimental import pallas as pl

K = 16
BM = 256


def _fused_kernel(p_blk_ref, p_all_ref, idx_ref, w_ref):
    pb = p_blk_ref[0]
    pa = p_all_ref[0]
    n = pa.shape[0]

    inner = jnp.dot(pb, pa.T, preferred_element_type=jnp.float32)
    x2b = jnp.sum(pb * pb, axis=1)
    x2a = jnp.sum(pa * pa, axis=1)
    d2 = x2b[:, None] + x2a[None, :] - 2.0 * inner
    d2 = jnp.maximum(d2, 1e-12)

    i = pl.program_id(1)
    row_ids = i * BM + jax.lax.broadcasted_iota(jnp.int32, (BM, n), 0)
    col_ids = jax.lax.broadcasted_iota(jnp.int32, (BM, n), 1)
    d2 = jnp.where(row_ids == col_ids, jnp.inf, d2)

    col_f = col_ids.astype(jnp.float32)
    nf = jnp.float32(n)

    vals = []
    idxs = []
    work = d2
    for _ in range(K):
        m = jnp.min(work, axis=1)
        hit = work == m[:, None]
        j = jnp.min(jnp.where(hit, col_f, nf), axis=1)
        work = jnp.where(col_f == j[:, None], jnp.inf, work)
        vals.append(m)
        idxs.append(j)

    v = jnp.stack(vals, axis=1)
    dist = jnp.sqrt(v)
    scores = -dist
    e = jnp.exp(scores - scores[:, :1])
    w = e / jnp.sum(e, axis=1, keepdims=True)

    idx_ref[0] = jnp.stack(idxs, axis=1).astype(jnp.int32)
    w_ref[0] = w


@jax.jit
def kernel(positions):
    batch, agents, dim = positions.shape
    grid = (batch, agents // BM)

    out_shape = (
        jax.ShapeDtypeStruct((batch, agents, K), jnp.int32),
        jax.ShapeDtypeStruct((batch, agents, K), jnp.float32),
    )
    idx, w = pl.pallas_call(
        _fused_kernel,
        grid=grid,
        in_specs=[
            pl.BlockSpec((1, BM, dim), lambda b, i: (b, i, 0)),
            pl.BlockSpec((1, agents, dim), lambda b, i: (b, 0, 0)),
        ],
        out_specs=[
            pl.BlockSpec((1, BM, K), lambda b, i: (b, i, 0)),
            pl.BlockSpec((1, BM, K), lambda b, i: (b, i, 0)),
        ],
        out_shape=out_shape,
    )(positions, positions)
    return idx, w


# speedup vs baseline: 12.3193x; 1.0002x over previous
import functools

import jax
import jax.numpy as jnp
from jax.experimental import pallas as pl
from jax.experimental.pallas import tpu as pltpu

K = 16
BM = 256


def _fused_kernel(p_blk_ref, p_all_ref, idx_ref, w_ref):
    pb = p_blk_ref[0]
    pa = p_all_ref[0]
    n = pa.shape[0]

    inner = jnp.dot(pb, pa.T, preferred_element_type=jnp.float32)
    x2b = jnp.sum(pb * pb, axis=1)
    x2a = jnp.sum(pa * pa, axis=1)
    d2 = x2b[:, None] + x2a[None, :] - 2.0 * inner
    d2 = jnp.maximum(d2, 1e-12)

    i = pl.program_id(1)
    row_ids = i * BM + jax.lax.broadcasted_iota(jnp.int32, (BM, n), 0)
    col_ids = jax.lax.broadcasted_iota(jnp.int32, (BM, n), 1)
    d2 = jnp.where(row_ids == col_ids, jnp.inf, d2)

    col_f = col_ids.astype(jnp.float32)
    nf = jnp.float32(n)

    vals = []
    idxs = []
    work = d2
    for k in range(K):
        m = jnp.min(work, axis=1)
        hit = work == m[:, None]
        j = jnp.min(jnp.where(hit, col_f, nf), axis=1)
        vals.append(m)
        idxs.append(j)
        if k < K - 1:
            work = jnp.where(col_f == j[:, None], jnp.inf, work)

    v = jnp.stack(vals, axis=1)
    dist = jnp.sqrt(v)
    scores = -dist
    e = jnp.exp(scores - scores[:, :1])
    w = e / jnp.sum(e, axis=1, keepdims=True)

    idx_ref[0] = jnp.stack(idxs, axis=1).astype(jnp.int32)
    w_ref[0] = w


@jax.jit
def kernel(positions):
    batch, agents, dim = positions.shape
    grid = (batch, agents // BM)

    out_shape = (
        jax.ShapeDtypeStruct((batch, agents, K), jnp.int32),
        jax.ShapeDtypeStruct((batch, agents, K), jnp.float32),
    )
    idx, w = pl.pallas_call(
        _fused_kernel,
        grid=grid,
        in_specs=[
            pl.BlockSpec((1, BM, dim), lambda b, i: (b, i, 0)),
            pl.BlockSpec((1, agents, dim), lambda b, i: (b, 0, 0)),
        ],
        out_specs=[
            pl.BlockSpec((1, BM, K), lambda b, i: (b, i, 0)),
            pl.BlockSpec((1, BM, K), lambda b, i: (b, i, 0)),
        ],
        out_shape=out_shape,
        compiler_params=pltpu.CompilerParams(
            dimension_semantics=("parallel", "parallel"),
        ),
    )(positions, positions)
    return idx, w


# speedup vs baseline: 12.8707x; 1.0448x over previous
import jax
import jax.numpy as jnp
from jax.experimental import pallas as pl
from jax.experimental.pallas import tpu as pltpu

K = 16
BM = 512


def _fused_kernel(p_blk_ref, p_all_ref, idx_ref, w_ref):
    pb = p_blk_ref[0]
    pa = p_all_ref[0]
    n = pa.shape[0]

    inner = jnp.dot(pb, pa.T, preferred_element_type=jnp.float32)
    x2b = jnp.sum(pb * pb, axis=1)
    x2a = jnp.sum(pa * pa, axis=1)
    d2 = x2b[:, None] + x2a[None, :] - 2.0 * inner
    d2 = jnp.maximum(d2, 1e-12)

    i = pl.program_id(1)
    row_ids = i * BM + jax.lax.broadcasted_iota(jnp.int32, (BM, n), 0)
    col_ids = jax.lax.broadcasted_iota(jnp.int32, (BM, n), 1)
    d2 = jnp.where(row_ids == col_ids, jnp.inf, d2)

    col_f = col_ids.astype(jnp.float32)
    nf = jnp.float32(n)

    vals = []
    idxs = []
    work = d2
    for k in range(K):
        m = jnp.min(work, axis=1)
        hit = work == m[:, None]
        j = jnp.min(jnp.where(hit, col_f, nf), axis=1)
        vals.append(m)
        idxs.append(j)
        if k < K - 1:
            work = jnp.where(col_f == j[:, None], jnp.inf, work)

    v = jnp.stack(vals, axis=1)
    dist = jnp.sqrt(v)
    scores = -dist
    e = jnp.exp(scores - scores[:, :1])
    w = e / jnp.sum(e, axis=1, keepdims=True)

    idx_ref[0] = jnp.stack(idxs, axis=1).astype(jnp.int32)
    w_ref[0] = w


@jax.jit
def kernel(positions):
    batch, agents, dim = positions.shape
    grid = (batch, agents // BM)

    out_shape = (
        jax.ShapeDtypeStruct((batch, agents, K), jnp.int32),
        jax.ShapeDtypeStruct((batch, agents, K), jnp.float32),
    )
    idx, w = pl.pallas_call(
        _fused_kernel,
        grid=grid,
        in_specs=[
            pl.BlockSpec((1, BM, dim), lambda b, i: (b, i, 0)),
            pl.BlockSpec((1, agents, dim), lambda b, i: (b, 0, 0)),
        ],
        out_specs=[
            pl.BlockSpec((1, BM, K), lambda b, i: (b, i, 0)),
            pl.BlockSpec((1, BM, K), lambda b, i: (b, i, 0)),
        ],
        out_shape=out_shape,
        compiler_params=pltpu.CompilerParams(
            dimension_semantics=("parallel", "parallel"),
        ),
    )(positions, positions)
    return idx, w
